# baseline (speedup 1.0000x reference)
import functools

import jax
import jax.numpy as jnp
from jax.experimental import pallas as pl
from jax.experimental.pallas import tpu as pltpu

_INTERPRET = False


def _silu(v):
    half = 0.5 * v
    return half * (jnp.tanh(half) + 1.0)


def _body(n, bi, bj, depth, *refs):
    bf = jnp.bfloat16
    f32 = jnp.float32
    h_ref, x_ref, nz_ref = refs[0], refs[1], refs[2]
    pnames = ('We1', 'be1', 'We2', 'be2', 'Wx1', 'bx1',
              'Wx2', 'bx2', 'Wh1', 'bh1', 'Wh2', 'bh2')

    def pref(l, mi, name):
        return refs[3 + (l * 2 + mi) * len(pnames) + pnames.index(name)]

    v_ref = refs[3 + depth * 2 * len(pnames)]
    (h_buf, xm_buf, xs_buf, macc, xam, xas, a_buf, b_buf,
     wa_s, wb_s, w2_s, wx1_s, wx2_s,
     wdm_s, wds_s, be1_s, be2_s, bx1_s, bx2_s,
     wh1_s, bh1_s, wh2_s, bh2_s) = refs[4 + depth * 2 * len(pnames):]

    l = pl.program_id(0)
    i = pl.program_id(1)
    j = pl.program_id(2)
    nj = pl.num_programs(2)
    hd = h_ref.shape[1]
    c2 = 2 * hd

    @pl.when((l == 0) & (i == 0) & (j == 0))
    def _init():
        h_buf[0, :, 0:hd] = h_ref[...]
        h_buf[0, :, hd:c2] = h_ref[...]
        xm_buf[0] = x_ref[...]
        xs_buf[0] = x_ref[...]
        for ll in range(depth):
            zb = jnp.zeros((hd, hd), bf)
            zf = jnp.zeros((hd, hd), f32)
            zr = jnp.zeros((1, hd), bf)
            for mi in range(2):
                lo, hi_ = mi * hd, (mi + 1) * hd
                oo, oh = (1 - mi) * hd, (2 - mi) * hd
                we1 = pref(ll, mi, 'We1')
                wa_s[ll, lo:hi_, lo:hi_] = we1[0:hd, :].astype(bf)
                wa_s[ll, lo:hi_, oo:oh] = zb
                wb_s[ll, lo:hi_, lo:hi_] = we1[hd:c2, :].astype(bf)
                wb_s[ll, lo:hi_, oo:oh] = zb
                w2_s[ll, lo:hi_, lo:hi_] = pref(ll, mi, 'We2')[...].astype(bf)
                w2_s[ll, lo:hi_, oo:oh] = zb
                wx1_s[ll, lo:hi_, lo:hi_] = pref(ll, mi, 'Wx1')[...].astype(bf)
                wx1_s[ll, lo:hi_, oo:oh] = zb
                wx2_s[ll, lo:hi_, mi:mi + 1] = pref(ll, mi, 'Wx2')[...].astype(bf)
                wx2_s[ll, oo:oh, mi:mi + 1] = jnp.zeros((hd, 1), bf)
                be1_s[ll, :, lo:hi_] = pref(ll, mi, 'be1')[...].astype(bf)
                be2_s[ll, :, lo:hi_] = pref(ll, mi, 'be2')[...].astype(bf)
                bx1_s[ll, :, lo:hi_] = pref(ll, mi, 'bx1')[...].astype(bf)
                bx2_s[ll, :, mi:mi + 1] = pref(ll, mi, 'bx2')[...]
                wh1 = pref(ll, mi, 'Wh1')
                wh1_s[ll, lo:hi_, lo:hi_] = wh1[0:hd, :]
                wh1_s[ll, lo:hi_, oo:oh] = zf
                wh1_s[ll, c2 + lo:c2 + hi_, lo:hi_] = wh1[hd:c2, :]
                wh1_s[ll, c2 + lo:c2 + hi_, oo:oh] = zf
                bh1_s[ll, :, lo:hi_] = pref(ll, mi, 'bh1')[...]
                wh2_s[ll, lo:hi_, lo:hi_] = pref(ll, mi, 'Wh2')[...]
                wh2_s[ll, lo:hi_, oo:oh] = zf
                bh2_s[ll, :, lo:hi_] = pref(ll, mi, 'bh2')[...]
            wdm_s[ll, :, 0:hd] = pref(ll, 0, 'We1')[c2:c2 + 1, :].astype(bf)
            wdm_s[ll, :, hd:c2] = zr
            wds_s[ll, :, 0:hd] = zr
            wds_s[ll, :, hd:c2] = pref(ll, 1, 'We1')[c2:c2 + 1, :].astype(bf)

    @pl.when(j == 0)
    def _zero():
        macc[...] = jnp.zeros_like(macc)
        xam[...] = jnp.zeros_like(xam)
        xas[...] = jnp.zeros_like(xas)

    @pl.when((i == 0) & (j == 0))
    def _project_nodes():
        hb = h_buf[l].astype(bf)
        a_buf[...] = (jnp.dot(hb, wa_s[l], preferred_element_type=f32)
                      + be1_s[l]).astype(bf)
        b_buf[...] = jnp.dot(hb, wb_s[l],
                             preferred_element_type=f32).astype(bf)

    lr = 1 - l
    hi = h_buf[l, pl.ds(i * bi, bi), :]
    xim = xm_buf[l, pl.ds(i * bi, bi), :]
    xjm = xm_buf[l, pl.ds(j * bj, bj), :]
    xis = xs_buf[l, pl.ds(i * bi, bi), :]
    xjs = xs_buf[l, pl.ds(j * bj, bj), :]

    a = a_buf[pl.ds(i * bi, bi), :]
    b = b_buf[pl.ds(j * bj, bj), :]

    def dist2(xi, xj):
        xi2 = jnp.sum(xi * xi, axis=1, keepdims=True)
        xj2 = jnp.sum(xj * xj, axis=1, keepdims=True)
        cross = jax.lax.dot_general(
            xi, xj, dimension_numbers=(((1,), (1,)), ((), ())),
            preferred_element_type=f32)
        return (xi2 + xj2.reshape(1, bj) - 2.0 * cross).astype(bf)

    d2m = dist2(xim, xjm)
    d2s = dist2(xis, xjs)

    t1 = (a[:, None, :] + b[None, :, :]
          + d2m[:, :, None] * wdm_s[l][None, :, :]
          + d2s[:, :, None] * wds_s[l][None, :, :])
    t1b = _silu(t1)

    m_f = jnp.dot(t1b.reshape(bi * bj, c2), w2_s[l],
                  preferred_element_type=f32).astype(bf)
    m_b = _silu(m_f + be2_s[l])
    wv_f = jnp.dot(m_b, wx1_s[l], preferred_element_type=f32).astype(bf)
    wv_b = _silu(wv_f + bx1_s[l])
    wsc3 = jax.lax.dot_general(
        wv_b.reshape(bi, bj, c2), wx2_s[l],
        dimension_numbers=(((2,), (0,)), ((), ())),
        preferred_element_type=f32)

    msum = jnp.sum(m_b.reshape(bi, bj, c2), axis=1, dtype=f32)
    diag_j = (i * bi) // bj

    @pl.when(j != diag_j)
    def _acc_offdiag():
        macc[...] += msum

    @pl.when(j == diag_j)
    def _acc_diag():
        b_diag = b_buf[pl.ds(i * bi, bi), :]
        td = _silu(a + b_diag)
        md = _silu(jnp.dot(td, w2_s[l],
                           preferred_element_type=f32).astype(bf) + be2_s[l])
        macc[...] += msum - md.astype(f32)

    one = jnp.ones((bj, 1), f32)
    xjm1 = jnp.concatenate([xjm, one], axis=1)
    xjs1 = jnp.concatenate([xjs, one], axis=1)
    p = jnp.concatenate([wsc3[:, :, 0:1] * xjm1[None, :, :],
                         wsc3[:, :, 1:2] * xjs1[None, :, :]], axis=2)
    s = jnp.sum(p, axis=1)
    xam[...] += s[:, 3:4] * xim - s[:, 0:3]
    xas[...] += s[:, 7:8] * xis - s[:, 4:7]

    @pl.when(j == nj - 1)
    def _epilogue():
        sxm = jnp.sum(xm_buf[l], axis=0, keepdims=True)
        sxs = jnp.sum(xs_buf[l], axis=0, keepdims=True)
        bx2r = bx2_s[l]
        bm = bx2r[0, 0] * (n * xim - sxm)
        bs = bx2r[0, 1] * (n * xis - sxs)
        xm_buf[lr, pl.ds(i * bi, bi), :] = xim + (xam[...] + bm) / (n - 1)
        xs_buf[lr, pl.ds(i * bi, bi), :] = xis + (xas[...] + bs) / (n - 1)
        nin = jnp.concatenate([hi, macc[...]], axis=1)
        hh = _silu(jnp.dot(nin, wh1_s[l],
                           preferred_element_type=f32) + bh1_s[l])
        h_new = hi + jnp.dot(hh, wh2_s[l],
                             preferred_element_type=f32) + bh2_s[l]
        h_buf[lr, pl.ds(i * bi, bi), :] = _silu(h_new)

    @pl.when((l == depth - 1) & (i == pl.num_programs(1) - 1) & (j == nj - 1))
    def _head():
        mu = xm_buf[lr] - x_ref[...]
        mu = mu - jnp.mean(mu, axis=0, keepdims=True)
        v_ref[...] = nz_ref[...] * jnp.exp(xs_buf[lr]) + mu


def kernel(h, x, params_mu, params_sigma, key):
    n, hd = h.shape
    depth = len(params_mu)
    bi, bj = 128, 256
    bf = jnp.bfloat16
    f32 = jnp.float32
    noise = jax.random.normal(jax.random.key(key), x.shape)

    def shaped(p, name):
        a = p[name]
        return a.reshape(1, -1) if a.ndim == 1 else a

    pnames = ('We1', 'be1', 'We2', 'be2', 'Wx1', 'bx1',
              'Wx2', 'bx2', 'Wh1', 'bh1', 'Wh2', 'bh2')
    ops = [h, x, noise]
    for l in range(depth):
        for p in (params_mu[l], params_sigma[l]):
            ops.extend(shaped(p, name) for name in pnames)

    def full_spec(arr):
        return pl.BlockSpec(arr.shape, lambda l, i, j, nd=arr.ndim: (0,) * nd)

    v = pl.pallas_call(
        functools.partial(_body, n, bi, bj, depth),
        grid=(depth, n // bi, n // bj),
        in_specs=[full_spec(o) for o in ops],
        out_specs=pl.BlockSpec((n, 3), lambda l, i, j: (0, 0)),
        out_shape=jax.ShapeDtypeStruct((n, 3), f32),
        scratch_shapes=[
            pltpu.VMEM((2, n, 2 * hd), f32),
            pltpu.VMEM((2, n, 3), f32),
            pltpu.VMEM((2, n, 3), f32),
            pltpu.VMEM((bi, 2 * hd), f32),
            pltpu.VMEM((bi, 3), f32),
            pltpu.VMEM((bi, 3), f32),
            pltpu.VMEM((n, 2 * hd), bf),
            pltpu.VMEM((n, 2 * hd), bf),
            pltpu.VMEM((depth, 2 * hd, 2 * hd), bf),
            pltpu.VMEM((depth, 2 * hd, 2 * hd), bf),
            pltpu.VMEM((depth, 2 * hd, 2 * hd), bf),
            pltpu.VMEM((depth, 2 * hd, 2 * hd), bf),
            pltpu.VMEM((depth, 2 * hd, 2), bf),
            pltpu.VMEM((depth, 1, 2 * hd), bf),
            pltpu.VMEM((depth, 1, 2 * hd), bf),
            pltpu.VMEM((depth, 1, 2 * hd), bf),
            pltpu.VMEM((depth, 1, 2 * hd), bf),
            pltpu.VMEM((depth, 1, 2 * hd), bf),
            pltpu.VMEM((depth, 1, 2), f32),
            pltpu.VMEM((depth, 4 * hd, 2 * hd), f32),
            pltpu.VMEM((depth, 1, 2 * hd), f32),
            pltpu.VMEM((depth, 2 * hd, 2 * hd), f32),
            pltpu.VMEM((depth, 1, 2 * hd), f32),
        ],
        compiler_params=pltpu.CompilerParams(
            dimension_semantics=("arbitrary", "arbitrary", "arbitrary")),
        interpret=_INTERPRET,
    )(*ops)
    return v


# speedup vs baseline: 1.2583x; 1.2583x over previous
import functools

import jax
import jax.numpy as jnp
from jax.experimental import pallas as pl
from jax.experimental.pallas import tpu as pltpu

_INTERPRET = False


def _silu(v):
    half = 0.5 * v
    return half * (jnp.tanh(half) + 1.0)


def _body(n, bi, bj, depth, *refs):
    bf = jnp.bfloat16
    f32 = jnp.float32
    h_ref, x_ref, nz_ref = refs[0], refs[1], refs[2]
    pnames = ('We1', 'be1', 'We2', 'be2', 'Wx1', 'bx1',
              'Wx2', 'bx2', 'Wh1', 'bh1', 'Wh2', 'bh2')

    def pref(l, mi, name):
        return refs[3 + (l * 2 + mi) * len(pnames) + pnames.index(name)]

    v_ref = refs[3 + depth * 2 * len(pnames)]
    (h_buf, xm_buf, xs_buf, macc, xam, xas, a_buf, b_buf,
     wa_s, wb_s, w2_s, wx1_s, wx2_s,
     wdm_s, wds_s, be1_s, be2_s, bx1_s, bx2_s,
     wh1_s, bh1_s, wh2_s, bh2_s) = refs[4 + depth * 2 * len(pnames):]

    l = pl.program_id(0)
    i = pl.program_id(1)
    j = pl.program_id(2)
    nj = pl.num_programs(2)
    hd = h_ref.shape[1]
    c2 = 2 * hd

    @pl.when((l == 0) & (i == 0) & (j == 0))
    def _init():
        h_buf[0, :, 0:hd] = h_ref[...]
        h_buf[0, :, hd:c2] = h_ref[...]
        xm_buf[0] = x_ref[...]
        xs_buf[0] = x_ref[...]
        for ll in range(depth):
            zb = jnp.zeros((hd, hd), bf)
            zf = jnp.zeros((hd, hd), f32)
            zr = jnp.zeros((1, hd), bf)
            for mi in range(2):
                lo, hi_ = mi * hd, (mi + 1) * hd
                oo, oh = (1 - mi) * hd, (2 - mi) * hd
                we1 = pref(ll, mi, 'We1')
                wa_s[ll, lo:hi_, lo:hi_] = we1[0:hd, :].astype(bf)
                wa_s[ll, lo:hi_, oo:oh] = zb
                wb_s[ll, lo:hi_, lo:hi_] = we1[hd:c2, :].astype(bf)
                wb_s[ll, lo:hi_, oo:oh] = zb
                w2_s[ll, lo:hi_, lo:hi_] = pref(ll, mi, 'We2')[...].astype(bf)
                w2_s[ll, lo:hi_, oo:oh] = zb
                wx1_s[ll, lo:hi_, lo:hi_] = pref(ll, mi, 'Wx1')[...].astype(bf)
                wx1_s[ll, lo:hi_, oo:oh] = zb
                wx2_s[ll, lo:hi_, mi:mi + 1] = pref(ll, mi, 'Wx2')[...].astype(bf)
                wx2_s[ll, oo:oh, mi:mi + 1] = jnp.zeros((hd, 1), bf)
                be1_s[ll, :, lo:hi_] = pref(ll, mi, 'be1')[...].astype(bf)
                be2_s[ll, :, lo:hi_] = pref(ll, mi, 'be2')[...].astype(bf)
                bx1_s[ll, :, lo:hi_] = pref(ll, mi, 'bx1')[...].astype(bf)
                bx2_s[ll, :, mi:mi + 1] = pref(ll, mi, 'bx2')[...]
                wh1 = pref(ll, mi, 'Wh1')
                wh1_s[ll, lo:hi_, lo:hi_] = wh1[0:hd, :]
                wh1_s[ll, lo:hi_, oo:oh] = zf
                wh1_s[ll, c2 + lo:c2 + hi_, lo:hi_] = wh1[hd:c2, :]
                wh1_s[ll, c2 + lo:c2 + hi_, oo:oh] = zf
                bh1_s[ll, :, lo:hi_] = pref(ll, mi, 'bh1')[...]
                wh2_s[ll, lo:hi_, lo:hi_] = pref(ll, mi, 'Wh2')[...]
                wh2_s[ll, lo:hi_, oo:oh] = zf
                bh2_s[ll, :, lo:hi_] = pref(ll, mi, 'bh2')[...]
            wdm_s[ll, :, 0:hd] = pref(ll, 0, 'We1')[c2:c2 + 1, :].astype(bf)
            wdm_s[ll, :, hd:c2] = zr
            wds_s[ll, :, 0:hd] = zr
            wds_s[ll, :, hd:c2] = pref(ll, 1, 'We1')[c2:c2 + 1, :].astype(bf)

    @pl.when(j == 0)
    def _zero():
        macc[...] = jnp.zeros_like(macc)
        xam[...] = jnp.zeros_like(xam)
        xas[...] = jnp.zeros_like(xas)

    @pl.when((i == 0) & (j == 0))
    def _project_nodes():
        hb = h_buf[l].astype(bf)
        a_buf[...] = (jnp.dot(hb, wa_s[l], preferred_element_type=f32)
                      + be1_s[l]).astype(bf)
        b_buf[...] = jnp.dot(hb, wb_s[l],
                             preferred_element_type=f32).astype(bf)

    lr = 1 - l
    hi = h_buf[l, pl.ds(i * bi, bi), :]
    xim = xm_buf[l, pl.ds(i * bi, bi), :]
    xjm = xm_buf[l, pl.ds(j * bj, bj), :]
    xis = xs_buf[l, pl.ds(i * bi, bi), :]
    xjs = xs_buf[l, pl.ds(j * bj, bj), :]

    a = a_buf[pl.ds(i * bi, bi), :]
    b = b_buf[pl.ds(j * bj, bj), :]

    def dist2(xi, xj):
        xi2 = jnp.sum(xi * xi, axis=1, keepdims=True)
        xj2 = jnp.sum(xj * xj, axis=1, keepdims=True)
        cross = jax.lax.dot_general(
            xi, xj, dimension_numbers=(((1,), (1,)), ((), ())),
            preferred_element_type=f32)
        return (xi2 + xj2.reshape(1, bj) - 2.0 * cross).astype(bf)

    d2m = dist2(xim, xjm)
    d2s = dist2(xis, xjs)

    t1 = (a[:, None, :] + b[None, :, :]
          + d2m[:, :, None] * wdm_s[l][None, :, :]
          + d2s[:, :, None] * wds_s[l][None, :, :])
    t1b = _silu(t1)

    m_f = jnp.dot(t1b.reshape(bi * bj, c2), w2_s[l],
                  preferred_element_type=f32).astype(bf)
    m_b = _silu(m_f + be2_s[l])
    wv_f = jnp.dot(m_b, wx1_s[l], preferred_element_type=f32).astype(bf)
    wv_b = _silu(wv_f + bx1_s[l])
    wsc3 = jax.lax.dot_general(
        wv_b.reshape(bi, bj, c2), wx2_s[l],
        dimension_numbers=(((2,), (0,)), ((), ())),
        preferred_element_type=f32)

    msum = jnp.sum(m_b.reshape(bi, bj, c2), axis=1, dtype=f32)
    diag_j = (i * bi) // bj

    @pl.when(j != diag_j)
    def _acc_offdiag():
        macc[...] += msum

    @pl.when(j == diag_j)
    def _acc_diag():
        b_diag = b_buf[pl.ds(i * bi, bi), :]
        td = _silu(a + b_diag)
        md = _silu(jnp.dot(td, w2_s[l],
                           preferred_element_type=f32).astype(bf) + be2_s[l])
        macc[...] += msum - md.astype(f32)

    wm = wsc3[:, :, 0]
    ws = wsc3[:, :, 1]
    rwm = jnp.sum(wm, axis=1, keepdims=True)
    rws = jnp.sum(ws, axis=1, keepdims=True)
    xam[...] += rwm * xim - jnp.dot(wm, xjm, preferred_element_type=f32)
    xas[...] += rws * xis - jnp.dot(ws, xjs, preferred_element_type=f32)

    @pl.when(j == nj - 1)
    def _epilogue():
        sxm = jnp.sum(xm_buf[l], axis=0, keepdims=True)
        sxs = jnp.sum(xs_buf[l], axis=0, keepdims=True)
        bx2r = bx2_s[l]
        bm = bx2r[0, 0] * (n * xim - sxm)
        bs = bx2r[0, 1] * (n * xis - sxs)
        xm_buf[lr, pl.ds(i * bi, bi), :] = xim + (xam[...] + bm) / (n - 1)
        xs_buf[lr, pl.ds(i * bi, bi), :] = xis + (xas[...] + bs) / (n - 1)
        nin = jnp.concatenate([hi, macc[...]], axis=1)
        hh = _silu(jnp.dot(nin, wh1_s[l],
                           preferred_element_type=f32) + bh1_s[l])
        h_new = hi + jnp.dot(hh, wh2_s[l],
                             preferred_element_type=f32) + bh2_s[l]
        h_buf[lr, pl.ds(i * bi, bi), :] = _silu(h_new)

    @pl.when((l == depth - 1) & (i == pl.num_programs(1) - 1) & (j == nj - 1))
    def _head():
        mu = xm_buf[lr] - x_ref[...]
        mu = mu - jnp.mean(mu, axis=0, keepdims=True)
        v_ref[...] = nz_ref[...] * jnp.exp(xs_buf[lr]) + mu


def kernel(h, x, params_mu, params_sigma, key):
    n, hd = h.shape
    depth = len(params_mu)
    bi, bj = 128, 256
    bf = jnp.bfloat16
    f32 = jnp.float32
    noise = jax.random.normal(jax.random.key(key), x.shape)

    def shaped(p, name):
        a = p[name]
        return a.reshape(1, -1) if a.ndim == 1 else a

    pnames = ('We1', 'be1', 'We2', 'be2', 'Wx1', 'bx1',
              'Wx2', 'bx2', 'Wh1', 'bh1', 'Wh2', 'bh2')
    ops = [h, x, noise]
    for l in range(depth):
        for p in (params_mu[l], params_sigma[l]):
            ops.extend(shaped(p, name) for name in pnames)

    def full_spec(arr):
        return pl.BlockSpec(arr.shape, lambda l, i, j, nd=arr.ndim: (0,) * nd)

    v = pl.pallas_call(
        functools.partial(_body, n, bi, bj, depth),
        grid=(depth, n // bi, n // bj),
        in_specs=[full_spec(o) for o in ops],
        out_specs=pl.BlockSpec((n, 3), lambda l, i, j: (0, 0)),
        out_shape=jax.ShapeDtypeStruct((n, 3), f32),
        scratch_shapes=[
            pltpu.VMEM((2, n, 2 * hd), f32),
            pltpu.VMEM((2, n, 3), f32),
            pltpu.VMEM((2, n, 3), f32),
            pltpu.VMEM((bi, 2 * hd), f32),
            pltpu.VMEM((bi, 3), f32),
            pltpu.VMEM((bi, 3), f32),
            pltpu.VMEM((n, 2 * hd), bf),
            pltpu.VMEM((n, 2 * hd), bf),
            pltpu.VMEM((depth, 2 * hd, 2 * hd), bf),
            pltpu.VMEM((depth, 2 * hd, 2 * hd), bf),
            pltpu.VMEM((depth, 2 * hd, 2 * hd), bf),
            pltpu.VMEM((depth, 2 * hd, 2 * hd), bf),
            pltpu.VMEM((depth, 2 * hd, 2), bf),
            pltpu.VMEM((depth, 1, 2 * hd), bf),
            pltpu.VMEM((depth, 1, 2 * hd), bf),
            pltpu.VMEM((depth, 1, 2 * hd), bf),
            pltpu.VMEM((depth, 1, 2 * hd), bf),
            pltpu.VMEM((depth, 1, 2 * hd), bf),
            pltpu.VMEM((depth, 1, 2), f32),
            pltpu.VMEM((depth, 4 * hd, 2 * hd), f32),
            pltpu.VMEM((depth, 1, 2 * hd), f32),
            pltpu.VMEM((depth, 2 * hd, 2 * hd), f32),
            pltpu.VMEM((depth, 1, 2 * hd), f32),
        ],
        compiler_params=pltpu.CompilerParams(
            dimension_semantics=("arbitrary", "arbitrary", "arbitrary")),
        interpret=_INTERPRET,
    )(*ops)
    return v
